# Initial kernel scaffold; baseline (speedup 1.0000x reference)
import jax
import jax.numpy as jnp
from jax.experimental import pallas as pl


def kernel(x, edge_index, W1, a_src1, a_dst1, b1, W2, a_src2, a_dst2, b2):
    raise NotImplementedError("write your pallas kernel here")


# speedup vs baseline: 18.0598x; 18.0598x over previous
import functools

import jax
import jax.numpy as jnp
from jax import lax
from jax.experimental import pallas as pl
from jax.experimental.pallas import tpu as pltpu
from jax.experimental.pallas import tpu_sc as plsc

N = 10000
NP = 10240
IN_CH = 128
HID = 32
HEADS = 4
E_REAL = 320000 + N

NC, NS, LANES = 2, 16, 16
NW = NC * NS

CH = 96
EPT = -(-E_REAL // (NW * CH)) * CH
EPAD = EPT * NW
NCHUNK = EPT // CH

ROWS_PT = NP // NS

ACC1_C = 144
ACC2_C = 48

_f32 = jnp.float32
_i32 = jnp.int32


def _leaky(v):
    return jnp.where(v >= 0.0, v, 0.2 * v)


def _tc1_body(x_ref, w1_ref, a1_ref, hp_ref, asad_ref, smax_ref):
    h = jnp.dot(x_ref[...], w1_ref[...], preferred_element_type=_f32)
    hp_ref[...] = h
    asad = jnp.dot(h, a1_ref[...], preferred_element_type=_f32)
    asad_ref[...] = asad
    m = jnp.max(asad, axis=0, keepdims=True)
    s = _leaky(m[:, 0:4] + m[:, 4:8])
    smax_ref[...] = jnp.concatenate([s, jnp.zeros((1, 12), _f32)], axis=1)


def _tc2_body(acc_ref, b1_ref, w2_ref, a2_ref, hp2_ref, asad2_ref, smax2_ref):
    p = acc_ref[0] + acc_ref[1]
    den = p[:, 128:132] + 1e-16
    den128 = jnp.concatenate(
        [jnp.broadcast_to(den[:, h:h + 1], (NP, HID)) for h in range(HEADS)],
        axis=1)
    h1 = p[:, :128] / den128 + b1_ref[...]
    h1 = jnp.where(h1 > 0.0, h1, jnp.exp(h1) - 1.0)
    h2 = jnp.dot(h1, w2_ref[...], preferred_element_type=_f32)
    hp2_ref[...] = h2
    asad2 = jnp.dot(h2, a2_ref[...], preferred_element_type=_f32)
    asad2_ref[...] = asad2
    m = jnp.max(asad2, axis=0, keepdims=True)
    s = _leaky(m[:, 0:1] + m[:, 1:2])
    smax2_ref[...] = jnp.concatenate([s, jnp.zeros((1, 15), _f32)], axis=1)


def _tc3_body(acc_ref, b2_ref, out_ref):
    p = acc_ref[0] + acc_ref[1]
    den = p[:, HID:HID + 1] + 1e-16
    out_ref[...] = p[:, :HID] / jnp.broadcast_to(den, (NP, HID)) + b2_ref[...]


@functools.lru_cache(maxsize=None)
def _mesh():
    return plsc.VectorSubcoreMesh(core_axis_name="c", subcore_axis_name="s",
                                  num_cores=NC, num_subcores=NS)


def _zero_rows(zbuf_v, acc_sh, row0, ncols):
    nfull = ROWS_PT // CH
    rem = ROWS_PT - nfull * CH
    for i in range(nfull):
        pltpu.sync_copy(zbuf_v, acc_sh.at[pl.ds(row0 + i * CH, CH)])
    if rem:
        pltpu.sync_copy(zbuf_v.at[pl.ds(0, rem)],
                        acc_sh.at[pl.ds(row0 + nfull * CH, rem)])


def _copy_out_rows(acc_sh, out_hbm, c, row0, ncols):
    nfull = ROWS_PT // CH
    rem = ROWS_PT - nfull * CH
    for i in range(nfull):
        pltpu.sync_copy(acc_sh.at[pl.ds(row0 + i * CH, CH)],
                        out_hbm.at[c, pl.ds(row0 + i * CH, CH)])
    if rem:
        pltpu.sync_copy(acc_sh.at[pl.ds(row0 + nfull * CH, rem)],
                        out_hbm.at[c, pl.ds(row0 + nfull * CH, rem)])


def _edge_kernel_body(nheads, hdim, acc_c,
                      hp_hbm, asad_hbm, smax_hbm, src_hbm, dst_hbm,
                      accp_hbm,
                      src_v, dst_v, hrows_v, wrows_v, asrows_v, adrows_v,
                      smax_v, acc_sh, sem):
    c = lax.axis_index("c")
    s = lax.axis_index("s")
    wid = c * NS + s

    zero16 = jnp.zeros((16,), _f32)

    def zb(r, carry):
        for j in range(acc_c // 16):
            wrows_v[r, pl.ds(j * 16, 16)] = zero16
        return carry
    lax.fori_loop(0, CH, zb, 0)

    _zero_rows(wrows_v, acc_sh, s * ROWS_PT, acc_c)

    pltpu.sync_copy(smax_hbm, smax_v)
    sv = smax_v[...]
    s_splat = [jnp.full((16,), sv[h], _f32) for h in range(nheads)]

    plsc.subcore_barrier()

    ebase = wid * EPT
    iota16 = lax.iota(_i32, 16)

    def chunk(ci, carry):
        base = ebase + ci * CH
        pltpu.sync_copy(src_hbm.at[pl.ds(base, CH)], src_v)
        pltpu.sync_copy(dst_hbm.at[pl.ds(base, CH)], dst_v)
        cp1 = pltpu.async_copy(hp_hbm.at[src_v], hrows_v, sem)
        cp2 = pltpu.async_copy(asad_hbm.at[src_v], asrows_v, sem)
        cp3 = pltpu.async_copy(asad_hbm.at[dst_v], adrows_v, sem)
        cp1.wait()
        cp2.wait()
        cp3.wait()
        for b in range(CH // 16):
            rows = iota16 + (b * 16)
            isrc = src_v[pl.ds(b * 16, 16)]
            idst = dst_v[pl.ds(b * 16, 16)]
            ws = []
            for h in range(nheads):
                a_s = plsc.load_gather(asrows_v, [rows, jnp.full((16,), h, _i32)])
                a_d = plsc.load_gather(adrows_v, [rows, jnp.full((16,), nheads + h, _i32)])
                w = jnp.exp(_leaky(a_s + a_d) - s_splat[h])
                ws.append(w)
                plsc.store_scatter(
                    wrows_v, [rows, jnp.full((16,), nheads * hdim + h, _i32)], w)
            for col in range(nheads * hdim):
                cvec = jnp.full((16,), col, _i32)
                vals = plsc.load_gather(hrows_v, [rows, cvec])
                plsc.store_scatter(wrows_v, [rows, cvec], vals * ws[col // hdim])
        pltpu.sync_copy(wrows_v, acc_sh.at[dst_v], add=True)
        return carry

    lax.fori_loop(0, NCHUNK, chunk, 0)

    plsc.subcore_barrier()
    _copy_out_rows(acc_sh, accp_hbm, c, s * ROWS_PT, acc_c)


@functools.lru_cache(maxsize=None)
def _make_edge_kernel(nheads, hdim, acc_c):
    return functools.partial(
        pl.kernel,
        out_type=jax.ShapeDtypeStruct((NC, NP, acc_c), _f32),
        mesh=_mesh(),
        scratch_types=[
            pltpu.VMEM((CH,), _i32),
            pltpu.VMEM((CH,), _i32),
            pltpu.VMEM((CH, nheads * hdim), _f32),
            pltpu.VMEM((CH, acc_c), _f32),
            pltpu.VMEM((CH, 16), _f32),
            pltpu.VMEM((CH, 16), _f32),
            pltpu.VMEM((16,), _f32),
            pltpu.VMEM_SHARED((NP, acc_c), _f32),
            pltpu.SemaphoreType.DMA,
        ],
        compiler_params=pltpu.CompilerParams(use_tc_tiling_on_sc=False,
                                             needs_layout_passes=False),
    )(functools.partial(_edge_kernel_body, nheads, hdim, acc_c))


_tc1 = pl.pallas_call(
    _tc1_body,
    out_shape=(jax.ShapeDtypeStruct((NP, IN_CH), _f32),
               jax.ShapeDtypeStruct((NP, 16), _f32),
               jax.ShapeDtypeStruct((1, 16), _f32)))

_tc2 = pl.pallas_call(
    _tc2_body,
    out_shape=(jax.ShapeDtypeStruct((NP, HID), _f32),
               jax.ShapeDtypeStruct((NP, 16), _f32),
               jax.ShapeDtypeStruct((1, 16), _f32)),
    compiler_params=pltpu.CompilerParams(vmem_limit_bytes=56 * 1024 * 1024))

_tc3 = pl.pallas_call(
    _tc3_body,
    out_shape=jax.ShapeDtypeStruct((NP, HID), _f32))


@jax.jit
def kernel(x, edge_index, W1, a_src1, a_dst1, b1, W2, a_src2, a_dst2, b2):
    xp = jnp.pad(x, ((0, NP - N), (0, 0)))

    eye = jnp.eye(HEADS, dtype=_f32)
    a_src_m = (eye[:, None, :] * a_src1[:, :, None]).reshape(HEADS * HID, HEADS)
    a_dst_m = (eye[:, None, :] * a_dst1[:, :, None]).reshape(HEADS * HID, HEADS)
    a1 = jnp.concatenate([a_src_m, a_dst_m], axis=1)
    a1 = jnp.pad(a1, ((0, 0), (0, 8)))
    a2 = jnp.concatenate([a_src2.T, a_dst2.T], axis=1)
    a2 = jnp.pad(a2, ((0, 0), (0, 14)))

    loop = jnp.arange(N, dtype=_i32)
    pad_idx = (N + (jnp.arange(EPAD - E_REAL) % (NP - N))).astype(_i32)
    src_p = jnp.concatenate([edge_index[0], loop, pad_idx])
    dst_p = jnp.concatenate([edge_index[1], loop, pad_idx])

    hp, asad1, smax1 = _tc1(xp, W1, a1)
    acc1 = _make_edge_kernel(HEADS, HID, ACC1_C)(
        hp, asad1, smax1.reshape(16), src_p, dst_p)

    hp2, asad2, smax2 = _tc2(acc1, b1.reshape(1, IN_CH), W2, a2)
    acc2 = _make_edge_kernel(1, HID, ACC2_C)(
        hp2, asad2, smax2.reshape(16), src_p, dst_p)

    out = _tc3(acc2, b2.reshape(1, HID))
    return out[:N]
